# baseline (speedup 1.0000x reference)
import functools
import math

import jax
import jax.numpy as jnp
import numpy as np
from jax.experimental import pallas as pl
from jax.experimental.pallas import tpu as pltpu

_MAX_L = 4


def _l_list(max_l):
    out = []
    for l in range(max_l + 1):
        for lx in range(l, -1, -1):
            for ly in range(l - lx, -1, -1):
                out.append((lx, ly, l - lx - ly))
    return out


def _weights():
    lst = _l_list(_MAX_L)
    pref = np.zeros((35,), np.float32)
    slot = np.full((35,), -1, np.int32)
    for i, (lx, ly, lz) in enumerate(lst):
        l = lx + ly + lz
        if l == 0:
            continue
        pref[i] = math.factorial(l) / (
            math.factorial(lx) * math.factorial(ly) * math.factorial(lz))
        slot[i] = l - 1
    return pref, slot


def _mat(C):
    pref, slot = _weights()
    A = np.zeros((35 * C, 4 * C), np.float32)
    for i in range(35):
        s = slot[i]
        if s < 0:
            continue
        for c in range(C):
            A[i * C + c, s * C + c] = pref[i]
    return A


def _body(x_ref, a_ref, o_ref):
    x = x_ref[...]
    sq = x * x
    y = jnp.dot(sq, a_ref[...], preferred_element_type=jnp.float32)
    o_ref[...] = jnp.concatenate([x[:, :8], y], axis=1)


def kernel(node_attr):
    N, R, L, C = node_attr.shape
    rows = N * R
    x2 = node_attr.reshape(rows, L * C)
    A = jnp.asarray(_mat(C))

    BN = 2000
    grid = rows // BN

    out2 = pl.pallas_call(
        _body,
        grid=(grid,),
        in_specs=[
            pl.BlockSpec((BN, L * C), lambda i: (i, 0)),
            pl.BlockSpec((L * C, 4 * C), lambda i: (0, 0)),
        ],
        out_specs=pl.BlockSpec((BN, (4 + 1) * C), lambda i: (i, 0)),
        out_shape=jax.ShapeDtypeStruct((rows, (4 + 1) * C), jnp.float32),
    )(x2, A)
    return out2.reshape(N, R, 4 + 1, C)
